# baseline (speedup 1.0000x reference)
import functools

import jax
import jax.numpy as jnp
from jax import lax
from jax.experimental import pallas as pl
from jax.experimental.pallas import tpu as pltpu
from jax.experimental.pallas import tpu_sc as plsc

_NUM_HEADS = 16
_AREA = 256
_N = _AREA * _AREA
_TABLE = 961 * _NUM_HEADS

_info = plsc.get_sparse_core_info()
_NC, _NS, _L = _info.num_cores, _info.num_subcores, _info.num_lanes
_NW = _NC * _NS
_CHUNK = _N // _NW
_GROUPS = _CHUNK // _L

_MESH = plsc.VectorSubcoreMesh(core_axis_name="c", subcore_axis_name="s")


@functools.partial(
    pl.kernel,
    mesh=_MESH,
    out_type=jax.ShapeDtypeStruct((_NUM_HEADS, _N), jnp.float32),
    scratch_types=[
        pltpu.VMEM((_TABLE,), jnp.float32),
        pltpu.VMEM((_CHUNK,), jnp.int32),
        pltpu.VMEM((_NUM_HEADS, _CHUNK), jnp.float32),
    ],
    compiler_params=pltpu.CompilerParams(needs_layout_passes=False),
)
def _rpb_gather(table_hbm, idx_hbm, out_hbm, table_v, idx_v, out_v):
    wid = lax.axis_index("s") * _NC + lax.axis_index("c")
    base = wid * _CHUNK
    pltpu.sync_copy(table_hbm, table_v)
    pltpu.sync_copy(idx_hbm.at[pl.ds(base, _CHUNK)], idx_v)

    @plsc.parallel_loop(0, _GROUPS, unroll=4)
    def _group(g):
        off = g * _L
        idxv = idx_v[pl.ds(off, _L)]
        flat = idxv * _NUM_HEADS
        for h in range(_NUM_HEADS):
            out_v[h, pl.ds(off, _L)] = plsc.load_gather(table_v, [flat + h])
    pltpu.sync_copy(out_v, out_hbm.at[:, pl.ds(base, _CHUNK)])


def kernel(relative_position_bias_table, relative_position_index):
    out = _rpb_gather(
        relative_position_bias_table.reshape(-1),
        relative_position_index.reshape(-1),
    )
    return out.reshape(_NUM_HEADS, _AREA, _AREA)


# speedup vs baseline: 1.0698x; 1.0698x over previous
import functools

import jax
import jax.numpy as jnp
from jax import lax
from jax.experimental import pallas as pl
from jax.experimental.pallas import tpu as pltpu
from jax.experimental.pallas import tpu_sc as plsc

_NUM_HEADS = 16
_AREA = 256
_N = _AREA * _AREA
_TABLE = 961 * _NUM_HEADS

_info = plsc.get_sparse_core_info()
_NC, _NS, _L = _info.num_cores, _info.num_subcores, _info.num_lanes
_NW = _NC * _NS
_CHUNK = _N // _NW
_GROUPS = _CHUNK // _L

_MESH = plsc.VectorSubcoreMesh(core_axis_name="c", subcore_axis_name="s")


@functools.partial(
    pl.kernel,
    mesh=_MESH,
    out_type=jax.ShapeDtypeStruct((_NUM_HEADS, _N), jnp.float32),
    scratch_types=[
        pltpu.VMEM((_TABLE,), jnp.float32),
        pltpu.VMEM((_CHUNK,), jnp.int32),
        pltpu.VMEM((_NUM_HEADS, _CHUNK), jnp.float32),
    ],
    compiler_params=pltpu.CompilerParams(needs_layout_passes=False),
)
def _rpb_gather(table_hbm, idx_hbm, out_hbm, table_v, idx_v, out_v):
    wid = lax.axis_index("s") * _NC + lax.axis_index("c")
    base = wid * _CHUNK
    pltpu.sync_copy(table_hbm, table_v)
    pltpu.sync_copy(idx_hbm.at[pl.ds(base, _CHUNK)], idx_v)

    @plsc.parallel_loop(0, _GROUPS, unroll=2)
    def _group(g):
        off = g * _L
        idxv = idx_v[pl.ds(off, _L)]
        flat = idxv * _NUM_HEADS
        for h in range(_NUM_HEADS):
            out_v[h, pl.ds(off, _L)] = plsc.load_gather(table_v, [flat + h])
    pltpu.sync_copy(out_v, out_hbm.at[:, pl.ds(base, _CHUNK)])


def kernel(relative_position_bias_table, relative_position_index):
    out = _rpb_gather(
        relative_position_bias_table.reshape(-1),
        relative_position_index.reshape(-1),
    )
    return out.reshape(_NUM_HEADS, _AREA, _AREA)


# speedup vs baseline: 1.0738x; 1.0037x over previous
import functools

import jax
import jax.numpy as jnp
from jax import lax
from jax.experimental import pallas as pl
from jax.experimental.pallas import tpu as pltpu
from jax.experimental.pallas import tpu_sc as plsc

_NUM_HEADS = 16
_AREA = 256
_N = _AREA * _AREA
_TABLE = 961 * _NUM_HEADS

_info = plsc.get_sparse_core_info()
_NC, _NS, _L = _info.num_cores, _info.num_subcores, _info.num_lanes
_NW = _NC * _NS
_CHUNK = _N // _NW
_GROUPS = _CHUNK // _L

_MESH = plsc.VectorSubcoreMesh(core_axis_name="c", subcore_axis_name="s")


@functools.partial(
    pl.kernel,
    mesh=_MESH,
    out_type=jax.ShapeDtypeStruct((_NUM_HEADS, _N), jnp.float32),
    scratch_types=[
        pltpu.VMEM((_TABLE,), jnp.float32),
        pltpu.VMEM((_CHUNK,), jnp.int32),
        pltpu.VMEM((_NUM_HEADS, _CHUNK), jnp.float32),
        pltpu.SemaphoreType.DMA,
        pltpu.SemaphoreType.DMA,
        pltpu.SemaphoreType.DMA,
    ],
    compiler_params=pltpu.CompilerParams(needs_layout_passes=False),
)
def _rpb_gather(table_hbm, idx_hbm, out_hbm, table_v, idx_v, out_v,
                sem_t, sem_i, sem_o):
    wid = lax.axis_index("s") * _NC + lax.axis_index("c")
    base = wid * _CHUNK
    ct = pltpu.async_copy(table_hbm, table_v, sem_t)
    ci = pltpu.async_copy(idx_hbm.at[pl.ds(base, _CHUNK)], idx_v, sem_i)
    ct.wait()
    ci.wait()

    quarter = _CHUNK // 4
    outs = []
    for q in range(4):
        @plsc.parallel_loop(q * (_GROUPS // 4), (q + 1) * (_GROUPS // 4),
                            unroll=2)
        def _group(g):
            off = g * _L
            idxv = idx_v[pl.ds(off, _L)]
            flat = idxv * _NUM_HEADS
            for h in range(_NUM_HEADS):
                out_v[h, pl.ds(off, _L)] = plsc.load_gather(
                    table_v, [flat + h])

        outs.append(pltpu.async_copy(
            out_v.at[:, pl.ds(q * quarter, quarter)],
            out_hbm.at[:, pl.ds(base + q * quarter, quarter)],
            sem_o,
        ))
    for c in outs:
        c.wait()


def kernel(relative_position_bias_table, relative_position_index):
    out = _rpb_gather(
        relative_position_bias_table.reshape(-1),
        relative_position_index.reshape(-1),
    )
    return out.reshape(_NUM_HEADS, _AREA, _AREA)
